# Initial kernel scaffold; baseline (speedup 1.0000x reference)
import jax
import jax.numpy as jnp
from jax.experimental import pallas as pl


def kernel(x, w1, b1, a1, w2, b2, a2, w3, b3):
    raise NotImplementedError("write your pallas kernel here")


# speedup vs baseline: 1.1653x; 1.1653x over previous
import functools

import jax
import jax.numpy as jnp
from jax.experimental import pallas as pl
from jax.experimental.pallas import tpu as pltpu


def _fused_se_kernel(x_ref, w1t_ref, b1_ref, a1_ref,
                     w2t_ref, b2_ref, a2_ref,
                     w3t_ref, b3_ref,
                     out_ref, *, inv_hw):
    x = x_ref[...]
    gap = jnp.sum(x.astype(jnp.float32), axis=-1) * inv_hw
    y = jnp.dot(gap, w1t_ref[...], preferred_element_type=jnp.float32) + b1_ref[...]
    y = jnp.where(y >= 0.0, y, a1_ref[...] * y)
    y = jnp.dot(y, w2t_ref[...], preferred_element_type=jnp.float32) + b2_ref[...]
    y = jnp.where(y >= 0.0, y, a2_ref[...] * y)
    y = jnp.dot(y, w3t_ref[...], preferred_element_type=jnp.float32) + b3_ref[...]
    gate = jax.nn.sigmoid(y).astype(x.dtype)
    out_ref[...] = x * gate[:, :, None]


def _pick_batch_tile(batch, max_bytes, row_bytes):
    tb = 1
    for d in range(1, batch + 1):
        if batch % d == 0 and d * row_bytes <= max_bytes:
            tb = d
    return tb


@jax.jit
def kernel(x, w1, b1, a1, w2, b2, a2, w3, b3):
    B, C, H, W = x.shape
    hw = H * W
    dtype_bytes = jnp.dtype(x.dtype).itemsize

    row_bytes = C * hw * dtype_bytes
    tb = _pick_batch_tile(B, 8 * 1024 * 1024, row_bytes)

    x_flat = x.reshape(B, C, hw)
    grid = (B // tb,)

    w1t = w1.T
    w2t = w2.T
    w3t = w3.T

    full2 = lambda shape: pl.BlockSpec(shape, lambda i: (0, 0))

    tile_bytes = tb * row_bytes
    weight_bytes = 3 * C * C * 4 + 5 * C * 4
    vmem_limit = int(min(100 * 2**20, 4 * tile_bytes + 2 * weight_bytes + 2**20))

    body = functools.partial(_fused_se_kernel, inv_hw=1.0 / float(hw))

    out = pl.pallas_call(
        body,
        out_shape=jax.ShapeDtypeStruct((B, C, hw), x.dtype),
        grid=grid,
        in_specs=[
            pl.BlockSpec((tb, C, hw), lambda i: (i, 0, 0)),
            full2((C, C)), full2((1, C)), full2((1, C)),
            full2((C, C)), full2((1, C)), full2((1, C)),
            full2((C, C)), full2((1, C)),
        ],
        out_specs=pl.BlockSpec((tb, C, hw), lambda i: (i, 0, 0)),
        compiler_params=pltpu.CompilerParams(
            dimension_semantics=("parallel",),
            vmem_limit_bytes=vmem_limit,
        ),
    )(
        x_flat,
        w1t, b1, a1,
        w2t, b2, a2,
        w3t, b3,
    )
    return out.reshape(B, C, H, W)


# speedup vs baseline: 1.1778x; 1.0108x over previous
import functools

import jax
import jax.numpy as jnp
from jax.experimental import pallas as pl
from jax.experimental.pallas import tpu as pltpu


def _se_pipeline_kernel(x_hbm, w1t_ref, b1_ref, a1_ref,
                        w2t_ref, b2_ref, a2_ref,
                        w3t_ref, b3_ref,
                        out_hbm,
                        in_bufs, out_bufs, in_sems, out_sems,
                        *, tb, n_chunks, nbuf, inv_hw):
    def start_in(slot, step):
        pltpu.make_async_copy(
            x_hbm.at[pl.ds(step * tb, tb)], in_bufs.at[slot],
            in_sems.at[slot]).start()

    def wait_in(slot):
        pltpu.make_async_copy(
            x_hbm.at[pl.ds(0, tb)], in_bufs.at[slot],
            in_sems.at[slot]).wait()

    def start_out(slot, step):
        pltpu.make_async_copy(
            out_bufs.at[slot], out_hbm.at[pl.ds(step * tb, tb)],
            out_sems.at[slot]).start()

    def wait_out(slot):
        pltpu.make_async_copy(
            out_bufs.at[slot], out_hbm.at[pl.ds(0, tb)],
            out_sems.at[slot]).wait()

    w1t = w1t_ref[...]
    w2t = w2t_ref[...]
    w3t = w3t_ref[...]
    b1 = b1_ref[...]
    b2 = b2_ref[...]
    b3 = b3_ref[...]
    a1 = a1_ref[...]
    a2 = a2_ref[...]

    for s in range(min(nbuf, n_chunks)):
        start_in(s, s)

    for step in range(n_chunks):
        slot = step % nbuf
        wait_in(slot)
        if step >= nbuf:
            wait_out(slot)
        x = in_bufs[slot]
        gap = jnp.sum(x, axis=-1) * inv_hw
        y = jnp.dot(gap, w1t, preferred_element_type=jnp.float32) + b1
        y = jnp.where(y >= 0.0, y, a1 * y)
        y = jnp.dot(y, w2t, preferred_element_type=jnp.float32) + b2
        y = jnp.where(y >= 0.0, y, a2 * y)
        y = jnp.dot(y, w3t, preferred_element_type=jnp.float32) + b3
        gate = jax.nn.sigmoid(y).astype(x.dtype)
        out_bufs[slot] = x * gate[:, :, None]
        start_out(slot, step)
        if step + nbuf < n_chunks:
            start_in(slot, step + nbuf)

    for s in range(min(nbuf, n_chunks)):
        wait_out((n_chunks - min(nbuf, n_chunks) + s) % nbuf)


@jax.jit
def kernel(x, w1, b1, a1, w2, b2, a2, w3, b3):
    B, C, H, W = x.shape
    hw = H * W

    tb = 2
    while B % tb:
        tb //= 2
    n_chunks = B // tb
    nbuf = min(4, n_chunks)

    x_flat = x.reshape(B, C, hw)

    w1t = w1.T
    w2t = w2.T
    w3t = w3.T

    vmem = lambda shape: pl.BlockSpec(shape, lambda: tuple(0 for _ in shape))
    any_spec = pl.BlockSpec(memory_space=pl.ANY)

    buf_bytes = 2 * nbuf * tb * C * hw * 4
    weight_bytes = 3 * C * C * 4 + 5 * C * 4
    vmem_limit = int(min(100 * 2**20, buf_bytes + 2 * weight_bytes + 2**20))

    body = functools.partial(
        _se_pipeline_kernel,
        tb=tb, n_chunks=n_chunks, nbuf=nbuf, inv_hw=1.0 / float(hw))

    out = pl.pallas_call(
        body,
        out_shape=jax.ShapeDtypeStruct((B, C, hw), x.dtype),
        in_specs=[
            any_spec,
            vmem((C, C)), vmem((1, C)), vmem((1, C)),
            vmem((C, C)), vmem((1, C)), vmem((1, C)),
            vmem((C, C)), vmem((1, C)),
        ],
        out_specs=any_spec,
        scratch_shapes=[
            pltpu.VMEM((nbuf, tb, C, hw), jnp.float32),
            pltpu.VMEM((nbuf, tb, C, hw), jnp.float32),
            pltpu.SemaphoreType.DMA((nbuf,)),
            pltpu.SemaphoreType.DMA((nbuf,)),
        ],
        compiler_params=pltpu.CompilerParams(
            vmem_limit_bytes=vmem_limit,
        ),
    )(
        x_flat,
        w1t, b1, a1,
        w2t, b2, a2,
        w3t, b3,
    )
    return out.reshape(B, C, H, W)
